# baseline (speedup 1.0000x reference)
import functools

import jax
import jax.numpy as jnp
from jax import lax
from jax.experimental import pallas as pl
from jax.experimental.pallas import tpu as pltpu
from jax.experimental.pallas import tpu_sc as plsc

N_USERS = 25000
N_NODES = 50000
DIM = 64
N_EDGES = 800000
HALF = 25000
ACC_ROWS = 25008
DUMMY = HALF
NC, NS = 2, 16
RPW = ACC_ROWS // NS
SEGE = 6400
NREG = N_EDGES // SEGE
REG = 6528
SUBC = 128
OCH = 40
NOCH = HALF // OCH
PAIRS = 4096
PPW = PAIRS // (NC * NS)

_mesh = plsc.VectorSubcoreMesh(core_axis_name="c", subcore_axis_name="s")
_params = pltpu.CompilerParams(use_tc_tiling_on_sc=False, needs_layout_passes=False)

_BCAST_DNUMS = lax.GatherDimensionNumbers(
    offset_dims=(), collapsed_slice_dims=(0,), start_index_map=(0,))


def _bcast(v, j):
    idx = jnp.full((16,), j, dtype=jnp.int32)
    return lax.gather(v, idx[:, None], _BCAST_DNUMS, (1,),
                      mode=lax.GatherScatterMode.PROMISE_IN_BOUNDS)


@functools.partial(
    pl.kernel,
    out_type=(
        jax.ShapeDtypeStruct((NC, NREG, REG // SUBC, 3 * SUBC), jnp.int32),
        jax.ShapeDtypeStruct((NREG * 16,), jnp.int32),
    ),
    mesh=_mesh,
    compiler_params=_params,
    scratch_types=[
        pltpu.VMEM((SEGE,), jnp.int32),
        pltpu.VMEM((SEGE,), jnp.int32),
        pltpu.VMEM((SEGE,), jnp.float32),
        pltpu.VMEM((REG,), jnp.int32),
        pltpu.VMEM((REG,), jnp.int32),
        pltpu.VMEM((REG,), jnp.float32),
        pltpu.VMEM((REG,), jnp.int32),
        pltpu.VMEM((REG,), jnp.int32),
        pltpu.VMEM((REG,), jnp.float32),
        pltpu.VMEM((REG // SUBC, 3 * SUBC), jnp.int32),
        pltpu.VMEM((REG // SUBC, 3 * SUBC), jnp.int32),
        pltpu.VMEM((16,), jnp.int32),
    ],
)
def _prep(src_hbm, dst_hbm, val_hbm, combc, counts,
          sgsrc, sgdst, sgval, cs0, cd0, cv0, cs1, cd1, cv1, ci0, ci1, cntbuf):
    c = lax.axis_index("c")
    s = lax.axis_index("s")
    w = s * NC + c
    lanes = lax.broadcasted_iota(jnp.int32, (16,), 0)
    zi = jnp.zeros((16,), jnp.int32)
    zf = jnp.zeros((16,), jnp.float32)
    di = jnp.full((16,), DUMMY, jnp.int32)
    nseg = (NREG - w + NC * NS - 1) // (NC * NS)

    def seg_body(tt, _):
        t = w + tt * (NC * NS)
        base = t * SEGE
        pltpu.sync_copy(src_hbm.at[pl.ds(base, SEGE)], sgsrc)
        pltpu.sync_copy(dst_hbm.at[pl.ds(base, SEGE)], sgdst)
        pltpu.sync_copy(val_hbm.at[pl.ds(base, SEGE)], sgval)

        def grp(g, carry):
            c0, c1 = carry
            gg = g * 16
            sv = sgsrc[pl.ds(gg, 16)]
            dv = sgdst[pl.ds(gg, 16)]
            vv = sgval[pl.ds(gg, 16)]
            m0 = dv < HALF
            m1 = jnp.logical_not(m0)
            plsc.store_compressed(cs0.at[pl.ds(c0, 16)], sv, mask=m0)
            plsc.store_compressed(cd0.at[pl.ds(c0, 16)], dv, mask=m0)
            plsc.store_compressed(cv0.at[pl.ds(c0, 16)], vv, mask=m0)
            plsc.store_compressed(cs1.at[pl.ds(c1, 16)], sv, mask=m1)
            plsc.store_compressed(cd1.at[pl.ds(c1, 16)], dv - HALF, mask=m1)
            plsc.store_compressed(cv1.at[pl.ds(c1, 16)], vv, mask=m1)
            pc0 = jnp.sum(m0.astype(jnp.int32))
            return (c0 + pc0, c1 + (16 - pc0))

        c0, c1 = lax.fori_loop(0, SEGE // 16, grp, (jnp.int32(0), jnp.int32(0)))

        for i in range(8):
            o = i * 16
            cs0[pl.ds(c0 + o, 16)] = zi
            cd0[pl.ds(c0 + o, 16)] = di
            cv0[pl.ds(c0 + o, 16)] = zf
            cs1[pl.ds(c1 + o, 16)] = zi
            cd1[pl.ds(c1 + o, 16)] = di
            cv1[pl.ds(c1 + o, 16)] = zf

        def repack(u, _):
            for q in range(8):
                qo = q * 16
                fo = pl.ds(u * SUBC + qo, 16)
                ci0[u, pl.ds(qo, 16)] = cs0[fo]
                ci0[u, pl.ds(SUBC + qo, 16)] = cd0[fo]
                ci0[u, pl.ds(2 * SUBC + qo, 16)] = plsc.bitcast(cv0[fo], jnp.int32)
                ci1[u, pl.ds(qo, 16)] = cs1[fo]
                ci1[u, pl.ds(SUBC + qo, 16)] = cd1[fo]
                ci1[u, pl.ds(2 * SUBC + qo, 16)] = plsc.bitcast(cv1[fo], jnp.int32)
            return 0

        lax.fori_loop(0, REG // SUBC, repack, 0)
        pltpu.sync_copy(ci0, combc.at[0, t])
        pltpu.sync_copy(ci1, combc.at[1, t])
        cntbuf[pl.ds(0, 16)] = jnp.where(
            lanes == 0, c0, jnp.where(lanes == 1, c1, 0))
        pltpu.sync_copy(cntbuf, counts.at[pl.ds(t * 16, 16)])
        return 0

    lax.fori_loop(0, nseg, seg_body, 0)


@functools.partial(
    pl.kernel,
    out_type=jax.ShapeDtypeStruct((N_NODES, DIM), jnp.float32),
    mesh=_mesh,
    compiler_params=_params,
    scratch_types=[
        pltpu.VMEM_SHARED((ACC_ROWS, DIM), jnp.float32),
        pltpu.VMEM((3, 3 * SUBC), jnp.int32),
        pltpu.VMEM((2, SUBC, DIM), jnp.float32),
        pltpu.VMEM((2, OCH, DIM), jnp.float32),
        pltpu.VMEM((2, OCH, DIM), jnp.float32),
        pltpu.VMEM((16,), jnp.int32),
        pltpu.SemaphoreType.DMA,
        pltpu.SemaphoreType.DMA,
        pltpu.SemaphoreType.DMA,
    ],
)
def _spmm(emb_in, combc, counts, zeros_hbm, out,
          acc, comb3, rows2, inb2, acb2, cbuf, gsem, ssem, isem):
    c = lax.axis_index("c")
    s = lax.axis_index("s")
    lanes = lax.broadcasted_iota(jnp.int32, (16,), 0)
    pltpu.sync_copy(zeros_hbm, acc.at[pl.ds(s * RPW, RPW)])
    plsc.subcore_barrier()

    nreg_w = (NREG - s + NS - 1) // NS

    def issue_loads(t, u, sl):
        pltpu.async_copy(combc.at[c, t, u], comb3.at[sl], isem)

    def wait_loads(t, sl):
        pltpu.make_async_copy(combc.at[c, t, 0], comb3.at[sl], isem).wait()

    def issue_gather(sl, b):
        pltpu.async_copy(emb_in.at[comb3.at[sl, pl.ds(0, SUBC)]], rows2.at[b], gsem)

    def wait_gather(b):
        pltpu.make_async_copy(emb_in.at[comb3.at[0, pl.ds(0, SUBC)]],
                              rows2.at[b], gsem).wait()

    def wait_scatter(b, sl):
        pltpu.make_async_copy(rows2.at[b],
                              acc.at[comb3.at[sl, pl.ds(SUBC, SUBC)]], ssem).wait()

    def reg_body(rr, _):
        t = s + rr * NS
        pltpu.sync_copy(counts.at[pl.ds(t * 16, 16)], cbuf)
        cv = cbuf[pl.ds(0, 16)]
        cnt = jnp.sum(jnp.where(lanes == c, cv, 0))
        nsub = (cnt + (SUBC - 1)) // SUBC

        @pl.when(nsub > 0)
        def _():
            issue_loads(t, 0, 0)
            wait_loads(t, 0)
            issue_gather(0, 0)

            @pl.when(nsub > 1)
            def _():
                issue_loads(t, 1, 1)

        def body(u, _):
            b = lax.rem(u, 2)
            nb = 1 - b
            sl = lax.rem(u, 3)
            wait_gather(b)

            @pl.when(u > 0)
            def _():
                wait_scatter(nb, lax.rem(u + 2, 3))

            @pl.when(u + 1 < nsub)
            def _():
                sl1 = lax.rem(u + 1, 3)
                wait_loads(t, sl1)
                issue_gather(sl1, nb)

            @pl.when(u + 2 < nsub)
            def _():
                issue_loads(t, u + 2, lax.rem(u + 2, 3))

            def grp(q, _):
                vv = plsc.bitcast(comb3[sl, pl.ds(2 * SUBC + q * 16, 16)],
                                  jnp.float32)
                for j in range(16):
                    vb = _bcast(vv, j)
                    r = q * 16 + j
                    for k in range(4):
                        ks = pl.ds(k * 16, 16)
                        rows2[b, r, ks] = rows2[b, r, ks] * vb
                return 0

            lax.fori_loop(0, SUBC // 16, grp, 0)
            pltpu.async_copy(rows2.at[b],
                             acc.at[comb3.at[sl, pl.ds(SUBC, SUBC)]], ssem,
                             add=True)
            return 0

        lax.fori_loop(0, nsub, body, 0)

        @pl.when(nsub > 0)
        def _():
            wait_scatter(lax.rem(nsub - 1, 2), lax.rem(nsub - 1, 3))

        return 0

    lax.fori_loop(0, nreg_w, reg_body, 0)
    plsc.subcore_barrier()

    nch = (NOCH - s + NS - 1) // NS

    def ch_rows(t):
        ch = s + t * NS
        rel0 = ch * OCH
        return rel0, c * HALF + rel0

    def issue_och(t, p):
        rel0, row0 = ch_rows(t)
        pltpu.async_copy(emb_in.at[pl.ds(row0, OCH)], inb2.at[p], gsem)
        pltpu.async_copy(acc.at[pl.ds(rel0, OCH)], acb2.at[p], isem)

    def wait_och(p):
        pltpu.make_async_copy(emb_in.at[pl.ds(0, OCH)], inb2.at[p], gsem).wait()
        pltpu.make_async_copy(acc.at[pl.ds(0, OCH)], acb2.at[p], isem).wait()

    issue_och(0, 0)

    def och_body(t, _):
        p = lax.rem(t, 2)
        np_ = 1 - p
        wait_och(p)

        @pl.when(t + 1 < nch)
        def _():
            issue_och(t + 1, np_)

        @pl.when(t > 0)
        def _():
            rel0, row0 = ch_rows(t - 1)
            pltpu.make_async_copy(acb2.at[np_], out.at[pl.ds(row0, OCH)],
                                  ssem).wait()

        def rowb(r, _):
            for k in range(4):
                ks = pl.ds(k * 16, 16)
                acb2[p, r, ks] = 0.2 * inb2[p, r, ks] + 0.8 * acb2[p, r, ks]
            return 0

        lax.fori_loop(0, OCH, rowb, 0)
        rel0, row0 = ch_rows(t)
        pltpu.async_copy(acb2.at[p], out.at[pl.ds(row0, OCH)], ssem)
        return 0

    lax.fori_loop(0, nch, och_body, 0)
    rel0, row0 = ch_rows(nch - 1)
    pltpu.make_async_copy(acb2.at[lax.rem(nch - 1, 2)],
                          out.at[pl.ds(row0, OCH)], ssem).wait()


@functools.partial(
    pl.kernel,
    out_type=jax.ShapeDtypeStruct((PAIRS,), jnp.float32),
    mesh=_mesh,
    compiler_params=_params,
    scratch_types=[
        pltpu.VMEM((PPW,), jnp.int32),
        pltpu.VMEM((PPW,), jnp.int32),
        pltpu.VMEM((PPW, DIM), jnp.float32),
        pltpu.VMEM((PPW, DIM), jnp.float32),
        pltpu.VMEM((PPW,), jnp.float32),
        pltpu.SemaphoreType.DMA,
    ],
)
def _pairdot(e0, e1, e2, e3, users, items, out,
             uidx, iidx, ubuf, ibuf, gbuf, sem):
    c = lax.axis_index("c")
    s = lax.axis_index("s")
    w = s * NC + c
    pb = w * PPW
    pltpu.sync_copy(users.at[pl.ds(pb, PPW)], uidx)
    pltpu.sync_copy(items.at[pl.ds(pb, PPW)], iidx)

    def addoff(q, _):
        sl = pl.ds(q * 16, 16)
        iidx[sl] = iidx[sl] + N_USERS
        return 0

    lax.fori_loop(0, PPW // 16, addoff, 0)

    pltpu.async_copy(e0.at[uidx], ubuf, sem).wait()
    pltpu.async_copy(e1.at[uidx], ubuf, sem, add=True).wait()
    pltpu.async_copy(e2.at[uidx], ubuf, sem, add=True).wait()
    pltpu.async_copy(e3.at[uidx], ubuf, sem, add=True).wait()
    pltpu.async_copy(e0.at[iidx], ibuf, sem).wait()
    pltpu.async_copy(e1.at[iidx], ibuf, sem, add=True).wait()
    pltpu.async_copy(e2.at[iidx], ibuf, sem, add=True).wait()
    pltpu.async_copy(e3.at[iidx], ibuf, sem, add=True).wait()

    lanes = lax.broadcasted_iota(jnp.int32, (16,), 0)

    def gloop(g, _):
        rows16 = lanes + g * 16

        def dloop(d, a):
            dd = jnp.full((16,), 0, jnp.int32) + d
            uu = plsc.load_gather(ubuf, [rows16, dd])
            ii = plsc.load_gather(ibuf, [rows16, dd])
            return a + uu * ii

        a = lax.fori_loop(0, DIM, dloop, jnp.zeros((16,), jnp.float32))
        gbuf[pl.ds(g * 16, 16)] = a * 0.0625
        return 0

    lax.fori_loop(0, PPW // 16, gloop, 0)
    pltpu.sync_copy(gbuf, out.at[pl.ds(pb, PPW)])


def kernel(users, items, edge_index, edge_vals, user_emb, item_emb):
    all0 = jnp.concatenate([user_emb, item_emb], axis=0)
    dst = edge_index[0]
    src = edge_index[1]
    zeros = jnp.zeros((RPW, DIM), jnp.float32)
    combc, counts = _prep(src, dst, edge_vals)
    e1 = _spmm(all0, combc, counts, zeros)
    e2 = _spmm(e1, combc, counts, zeros)
    e3 = _spmm(e2, combc, counts, zeros)
    return _pairdot(all0, e1, e2, e3, users, items)


# speedup vs baseline: 1.0588x; 1.0588x over previous
import functools

import jax
import jax.numpy as jnp
from jax import lax
from jax.experimental import pallas as pl
from jax.experimental.pallas import tpu as pltpu
from jax.experimental.pallas import tpu_sc as plsc

N_USERS = 25000
N_NODES = 50000
DIM = 64
N_EDGES = 800000
HALF = 25000
ACC_ROWS = 25008
DUMMY = HALF
NC, NS = 2, 16
RPW = ACC_ROWS // NS
SEGE = 6400
NREG = N_EDGES // SEGE
REG = 6528
SUBC = 128
OCH = 25
NOCH = HALF // OCH
PAIRS = 4096
PPW = PAIRS // (NC * NS)

_mesh = plsc.VectorSubcoreMesh(core_axis_name="c", subcore_axis_name="s")
_params = pltpu.CompilerParams(use_tc_tiling_on_sc=False, needs_layout_passes=False)

_BCAST_DNUMS = lax.GatherDimensionNumbers(
    offset_dims=(), collapsed_slice_dims=(0,), start_index_map=(0,))


def _bcast(v, j):
    idx = jnp.full((16,), j, dtype=jnp.int32)
    return lax.gather(v, idx[:, None], _BCAST_DNUMS, (1,),
                      mode=lax.GatherScatterMode.PROMISE_IN_BOUNDS)


@functools.partial(
    pl.kernel,
    out_type=(
        jax.ShapeDtypeStruct((NC, NREG, REG // SUBC, 3 * SUBC), jnp.int32),
        jax.ShapeDtypeStruct((NREG * 16,), jnp.int32),
    ),
    mesh=_mesh,
    compiler_params=_params,
    scratch_types=[
        pltpu.VMEM((SEGE,), jnp.int32),
        pltpu.VMEM((SEGE,), jnp.int32),
        pltpu.VMEM((SEGE,), jnp.float32),
        pltpu.VMEM((REG,), jnp.int32),
        pltpu.VMEM((REG,), jnp.int32),
        pltpu.VMEM((REG,), jnp.float32),
        pltpu.VMEM((REG,), jnp.int32),
        pltpu.VMEM((REG,), jnp.int32),
        pltpu.VMEM((REG,), jnp.float32),
        pltpu.VMEM((REG // SUBC, 3 * SUBC), jnp.int32),
        pltpu.VMEM((REG // SUBC, 3 * SUBC), jnp.int32),
        pltpu.VMEM((16,), jnp.int32),
    ],
)
def _prep(src_hbm, dst_hbm, val_hbm, combc, counts,
          sgsrc, sgdst, sgval, cs0, cd0, cv0, cs1, cd1, cv1, ci0, ci1, cntbuf):
    c = lax.axis_index("c")
    s = lax.axis_index("s")
    w = s * NC + c
    lanes = lax.broadcasted_iota(jnp.int32, (16,), 0)
    zi = jnp.zeros((16,), jnp.int32)
    zf = jnp.zeros((16,), jnp.float32)
    di = jnp.full((16,), DUMMY, jnp.int32)
    nseg = (NREG - w + NC * NS - 1) // (NC * NS)

    def seg_body(tt, _):
        t = w + tt * (NC * NS)
        base = t * SEGE
        pltpu.sync_copy(src_hbm.at[pl.ds(base, SEGE)], sgsrc)
        pltpu.sync_copy(dst_hbm.at[pl.ds(base, SEGE)], sgdst)
        pltpu.sync_copy(val_hbm.at[pl.ds(base, SEGE)], sgval)

        def grp(g, carry):
            c0, c1 = carry
            gg = g * 16
            sv = sgsrc[pl.ds(gg, 16)]
            dv = sgdst[pl.ds(gg, 16)]
            vv = sgval[pl.ds(gg, 16)]
            m0 = dv < HALF
            m1 = jnp.logical_not(m0)
            plsc.store_compressed(cs0.at[pl.ds(c0, 16)], sv, mask=m0)
            plsc.store_compressed(cd0.at[pl.ds(c0, 16)], dv, mask=m0)
            plsc.store_compressed(cv0.at[pl.ds(c0, 16)], vv, mask=m0)
            plsc.store_compressed(cs1.at[pl.ds(c1, 16)], sv, mask=m1)
            plsc.store_compressed(cd1.at[pl.ds(c1, 16)], dv - HALF, mask=m1)
            plsc.store_compressed(cv1.at[pl.ds(c1, 16)], vv, mask=m1)
            pc0 = jnp.sum(m0.astype(jnp.int32))
            return (c0 + pc0, c1 + (16 - pc0))

        c0, c1 = lax.fori_loop(0, SEGE // 16, grp, (jnp.int32(0), jnp.int32(0)))

        for i in range(8):
            o = i * 16
            cs0[pl.ds(c0 + o, 16)] = zi
            cd0[pl.ds(c0 + o, 16)] = di
            cv0[pl.ds(c0 + o, 16)] = zf
            cs1[pl.ds(c1 + o, 16)] = zi
            cd1[pl.ds(c1 + o, 16)] = di
            cv1[pl.ds(c1 + o, 16)] = zf

        def repack(u, _):
            for q in range(8):
                qo = q * 16
                fo = pl.ds(u * SUBC + qo, 16)
                ci0[u, pl.ds(qo, 16)] = cs0[fo]
                ci0[u, pl.ds(SUBC + qo, 16)] = cd0[fo]
                ci0[u, pl.ds(2 * SUBC + qo, 16)] = plsc.bitcast(cv0[fo], jnp.int32)
                ci1[u, pl.ds(qo, 16)] = cs1[fo]
                ci1[u, pl.ds(SUBC + qo, 16)] = cd1[fo]
                ci1[u, pl.ds(2 * SUBC + qo, 16)] = plsc.bitcast(cv1[fo], jnp.int32)
            return 0

        lax.fori_loop(0, REG // SUBC, repack, 0)
        pltpu.sync_copy(ci0, combc.at[0, t])
        pltpu.sync_copy(ci1, combc.at[1, t])
        cntbuf[pl.ds(0, 16)] = jnp.where(
            lanes == 0, c0, jnp.where(lanes == 1, c1, 0))
        pltpu.sync_copy(cntbuf, counts.at[pl.ds(t * 16, 16)])
        return 0

    lax.fori_loop(0, nseg, seg_body, 0)


@functools.partial(
    pl.kernel,
    out_type=jax.ShapeDtypeStruct((N_NODES, DIM), jnp.float32),
    mesh=_mesh,
    compiler_params=_params,
    scratch_types=[
        pltpu.VMEM_SHARED((ACC_ROWS, DIM), jnp.float32),
        pltpu.VMEM((4, 3 * SUBC), jnp.int32),
        pltpu.VMEM((3, SUBC, DIM), jnp.float32),
        pltpu.VMEM((OCH, DIM), jnp.float32),
        pltpu.VMEM((OCH, DIM), jnp.float32),
        pltpu.VMEM((16,), jnp.int32),
        pltpu.SemaphoreType.DMA,
        pltpu.SemaphoreType.DMA,
        pltpu.SemaphoreType.DMA,
    ],
)
def _spmm(emb_in, combc, counts, zeros_hbm, out,
          acc, comb4, rows3, inb1, acb1, cbuf, gsem, ssem, isem):
    c = lax.axis_index("c")
    s = lax.axis_index("s")
    lanes = lax.broadcasted_iota(jnp.int32, (16,), 0)
    pltpu.sync_copy(zeros_hbm, acc.at[pl.ds(s * RPW, RPW)])
    plsc.subcore_barrier()

    nreg_w = (NREG - s + NS - 1) // NS

    def issue_loads(t, u, sl):
        pltpu.async_copy(combc.at[c, t, u], comb4.at[sl], isem)

    def wait_loads(t, sl):
        pltpu.make_async_copy(combc.at[c, t, 0], comb4.at[sl], isem).wait()

    def issue_gather(sl, b):
        pltpu.async_copy(emb_in.at[comb4.at[sl, pl.ds(0, SUBC)]], rows3.at[b], gsem)

    def wait_gather(b):
        pltpu.make_async_copy(emb_in.at[comb4.at[0, pl.ds(0, SUBC)]],
                              rows3.at[b], gsem).wait()

    def wait_scatter(b, sl):
        pltpu.make_async_copy(rows3.at[b],
                              acc.at[comb4.at[sl, pl.ds(SUBC, SUBC)]], ssem).wait()

    def reg_body(rr, _):
        t = s + rr * NS
        pltpu.sync_copy(counts.at[pl.ds(t * 16, 16)], cbuf)
        cv = cbuf[pl.ds(0, 16)]
        cnt = jnp.sum(jnp.where(lanes == c, cv, 0))
        nsub = (cnt + (SUBC - 1)) // SUBC

        @pl.when(nsub > 0)
        def _():
            issue_loads(t, 0, 0)
            wait_loads(t, 0)
            issue_gather(0, 0)

            @pl.when(nsub > 1)
            def _():
                issue_loads(t, 1, 1)

        def body(u, _):
            b = lax.rem(u, 3)
            sl = lax.rem(u, 4)
            wait_gather(b)

            @pl.when(u > 1)
            def _():
                wait_scatter(lax.rem(u + 1, 3), lax.rem(u + 2, 4))

            @pl.when(u + 1 < nsub)
            def _():
                wait_loads(t, lax.rem(u + 1, 4))
                issue_gather(lax.rem(u + 1, 4), lax.rem(u + 1, 3))

            @pl.when(u + 2 < nsub)
            def _():
                issue_loads(t, u + 2, lax.rem(u + 2, 4))

            def grp(q, _):
                vv = plsc.bitcast(comb4[sl, pl.ds(2 * SUBC + q * 16, 16)],
                                  jnp.float32)
                for j in range(16):
                    vb = _bcast(vv, j)
                    r = q * 16 + j
                    for k in range(4):
                        ks = pl.ds(k * 16, 16)
                        rows3[b, r, ks] = rows3[b, r, ks] * vb
                return 0

            lax.fori_loop(0, SUBC // 16, grp, 0)
            pltpu.async_copy(rows3.at[b],
                             acc.at[comb4.at[sl, pl.ds(SUBC, SUBC)]], ssem,
                             add=True)
            return 0

        lax.fori_loop(0, nsub, body, 0)

        @pl.when(nsub > 1)
        def _():
            wait_scatter(lax.rem(nsub - 2, 3), lax.rem(nsub - 2, 4))

        @pl.when(nsub > 0)
        def _():
            wait_scatter(lax.rem(nsub - 1, 3), lax.rem(nsub - 1, 4))

        return 0

    lax.fori_loop(0, nreg_w, reg_body, 0)
    plsc.subcore_barrier()

    nch = (NOCH - s + NS - 1) // NS

    def och_body(t, _):
        ch = s + t * NS
        rel0 = ch * OCH
        row0 = c * HALF + rel0
        pltpu.sync_copy(emb_in.at[pl.ds(row0, OCH)], inb1)
        pltpu.sync_copy(acc.at[pl.ds(rel0, OCH)], acb1)

        def rowb(r, _):
            for k in range(4):
                ks = pl.ds(k * 16, 16)
                acb1[r, ks] = 0.2 * inb1[r, ks] + 0.8 * acb1[r, ks]
            return 0

        lax.fori_loop(0, OCH, rowb, 0)
        pltpu.sync_copy(acb1, out.at[pl.ds(row0, OCH)])
        return 0

    lax.fori_loop(0, nch, och_body, 0)


@functools.partial(
    pl.kernel,
    out_type=jax.ShapeDtypeStruct((PAIRS,), jnp.float32),
    mesh=_mesh,
    compiler_params=_params,
    scratch_types=[
        pltpu.VMEM((PPW,), jnp.int32),
        pltpu.VMEM((PPW,), jnp.int32),
        pltpu.VMEM((PPW, DIM), jnp.float32),
        pltpu.VMEM((PPW, DIM), jnp.float32),
        pltpu.VMEM((PPW,), jnp.float32),
        pltpu.SemaphoreType.DMA,
    ],
)
def _pairdot(e0, e1, e2, e3, users, items, out,
             uidx, iidx, ubuf, ibuf, gbuf, sem):
    c = lax.axis_index("c")
    s = lax.axis_index("s")
    w = s * NC + c
    pb = w * PPW
    pltpu.sync_copy(users.at[pl.ds(pb, PPW)], uidx)
    pltpu.sync_copy(items.at[pl.ds(pb, PPW)], iidx)

    def addoff(q, _):
        sl = pl.ds(q * 16, 16)
        iidx[sl] = iidx[sl] + N_USERS
        return 0

    lax.fori_loop(0, PPW // 16, addoff, 0)

    pltpu.async_copy(e0.at[uidx], ubuf, sem).wait()
    pltpu.async_copy(e1.at[uidx], ubuf, sem, add=True).wait()
    pltpu.async_copy(e2.at[uidx], ubuf, sem, add=True).wait()
    pltpu.async_copy(e3.at[uidx], ubuf, sem, add=True).wait()
    pltpu.async_copy(e0.at[iidx], ibuf, sem).wait()
    pltpu.async_copy(e1.at[iidx], ibuf, sem, add=True).wait()
    pltpu.async_copy(e2.at[iidx], ibuf, sem, add=True).wait()
    pltpu.async_copy(e3.at[iidx], ibuf, sem, add=True).wait()

    lanes = lax.broadcasted_iota(jnp.int32, (16,), 0)

    def gloop(g, _):
        rows16 = lanes + g * 16

        def dloop(d, a):
            dd = jnp.full((16,), 0, jnp.int32) + d
            uu = plsc.load_gather(ubuf, [rows16, dd])
            ii = plsc.load_gather(ibuf, [rows16, dd])
            return a + uu * ii

        a = lax.fori_loop(0, DIM, dloop, jnp.zeros((16,), jnp.float32))
        gbuf[pl.ds(g * 16, 16)] = a * 0.0625
        return 0

    lax.fori_loop(0, PPW // 16, gloop, 0)
    pltpu.sync_copy(gbuf, out.at[pl.ds(pb, PPW)])


def kernel(users, items, edge_index, edge_vals, user_emb, item_emb):
    all0 = jnp.concatenate([user_emb, item_emb], axis=0)
    dst = edge_index[0]
    src = edge_index[1]
    zeros = jnp.zeros((RPW, DIM), jnp.float32)
    combc, counts = _prep(src, dst, edge_vals)
    e1 = _spmm(all0, combc, counts, zeros)
    e2 = _spmm(e1, combc, counts, zeros)
    e3 = _spmm(e2, combc, counts, zeros)
    return _pairdot(all0, e1, e2, e3, users, items)


# speedup vs baseline: 1.0673x; 1.0080x over previous
import functools

import jax
import jax.numpy as jnp
from jax import lax
from jax.experimental import pallas as pl
from jax.experimental.pallas import tpu as pltpu
from jax.experimental.pallas import tpu_sc as plsc

N_USERS = 25000
N_NODES = 50000
DIM = 64
N_EDGES = 800000
HALF = 25000
ACC_ROWS = 25008
DUMMY = HALF
NC, NS = 2, 16
EPW = N_EDGES // NS
OUTER = 2000
SUB = 80
RPW = ACC_ROWS // NS
OCH = 50
NSUB = EPW // SUB
SPO = OUTER // SUB
NOCH = HALF // OCH
PAIRS = 4096
PPW = PAIRS // (NC * NS)

_mesh = plsc.VectorSubcoreMesh(core_axis_name="c", subcore_axis_name="s")

_BCAST_DNUMS = lax.GatherDimensionNumbers(
    offset_dims=(), collapsed_slice_dims=(0,), start_index_map=(0,))


def _bcast(v, j):
    idx = jnp.full((16,), j, dtype=jnp.int32)
    return lax.gather(v, idx[:, None], _BCAST_DNUMS, (1,),
                      mode=lax.GatherScatterMode.PROMISE_IN_BOUNDS)


@functools.partial(
    pl.kernel,
    out_type=jax.ShapeDtypeStruct((N_NODES, DIM), jnp.float32),
    mesh=_mesh,
    compiler_params=pltpu.CompilerParams(use_tc_tiling_on_sc=False, needs_layout_passes=False),
    scratch_types=[
        pltpu.VMEM_SHARED((ACC_ROWS, DIM), jnp.float32),
        pltpu.VMEM((OUTER,), jnp.int32),
        pltpu.VMEM((OUTER,), jnp.int32),
        pltpu.VMEM((OUTER,), jnp.float32),
        pltpu.VMEM((2, SUB), jnp.int32),
        pltpu.VMEM((2, SUB, DIM), jnp.float32),
        pltpu.VMEM((OCH, DIM), jnp.float32),
        pltpu.VMEM((OCH, DIM), jnp.float32),
        pltpu.SemaphoreType.DMA,
        pltpu.SemaphoreType.DMA,
    ],
)
def _spmm(emb_in, src_hbm, dst_hbm, val_hbm, zeros_hbm, out,
          acc, srcst, dstst, valst, idx2, rows2, inbuf, accbuf, gsem, ssem):
    c = lax.axis_index("c")
    s = lax.axis_index("s")
    pltpu.sync_copy(zeros_hbm, acc.at[pl.ds(s * RPW, RPW)])
    plsc.subcore_barrier()

    base_w = s * EPW

    def stage(o):
        ob = base_w + o * OUTER
        pltpu.sync_copy(src_hbm.at[pl.ds(ob, OUTER)], srcst)
        pltpu.sync_copy(dst_hbm.at[pl.ds(ob, OUTER)], dstst)
        pltpu.sync_copy(val_hbm.at[pl.ds(ob, OUTER)], valst)

    def issue_gather(u, b):
        w0 = lax.rem(u, SPO) * SUB
        pltpu.async_copy(emb_in.at[srcst.at[pl.ds(w0, SUB)]], rows2.at[b], gsem)

    def wait_gather(b):
        pltpu.make_async_copy(emb_in.at[srcst.at[pl.ds(0, SUB)]],
                              rows2.at[b], gsem).wait()

    def wait_scatter(b):
        pltpu.make_async_copy(rows2.at[b], acc.at[idx2.at[b]], ssem).wait()

    stage(0)
    issue_gather(0, 0)

    def body(u, _):
        b = lax.rem(u, 2)
        nb = 1 - b
        wait_gather(b)
        boundary = lax.rem(u + 1, SPO) == 0
        notlast = u + 1 < NSUB

        @pl.when(u > 0)
        def _():
            wait_scatter(nb)

        @pl.when(notlast & jnp.logical_not(boundary))
        def _():
            issue_gather(u + 1, nb)

        w0 = lax.rem(u, SPO) * SUB

        def grp(q, _):
            go = w0 + q * 16
            dv = dstst[pl.ds(go, 16)]
            rel = dv - c * HALF
            ok = (rel >= 0) & (rel < HALF)
            idx2[b, pl.ds(q * 16, 16)] = jnp.where(ok, rel, DUMMY)
            vv = valst[pl.ds(go, 16)]
            for j in range(16):
                vb = _bcast(vv, j)
                r = q * 16 + j
                for k in range(4):
                    sl = pl.ds(k * 16, 16)
                    rows2[b, r, sl] = rows2[b, r, sl] * vb
            return 0

        lax.fori_loop(0, SUB // 16, grp, 0)

        @pl.when(boundary & notlast)
        def _():
            stage((u + 1) // SPO)
            issue_gather(u + 1, nb)

        pltpu.async_copy(rows2.at[b], acc.at[idx2.at[b]], ssem, add=True)
        return 0

    lax.fori_loop(0, NSUB, body, 0)
    wait_scatter((NSUB - 1) % 2)
    plsc.subcore_barrier()

    nch = (NOCH - s + NS - 1) // NS

    def och_body(t, _):
        ch = s + t * NS
        rel0 = ch * OCH
        row0 = c * HALF + rel0
        pltpu.sync_copy(emb_in.at[pl.ds(row0, OCH)], inbuf)
        pltpu.sync_copy(acc.at[pl.ds(rel0, OCH)], accbuf)

        def rowb(r, _):
            for k in range(4):
                sl = pl.ds(k * 16, 16)
                accbuf[r, sl] = 0.2 * inbuf[r, sl] + 0.8 * accbuf[r, sl]
            return 0

        lax.fori_loop(0, OCH, rowb, 0)
        pltpu.sync_copy(accbuf, out.at[pl.ds(row0, OCH)])
        return 0

    lax.fori_loop(0, nch, och_body, 0)


@functools.partial(
    pl.kernel,
    out_type=jax.ShapeDtypeStruct((PAIRS,), jnp.float32),
    mesh=_mesh,
    compiler_params=pltpu.CompilerParams(use_tc_tiling_on_sc=False, needs_layout_passes=False),
    scratch_types=[
        pltpu.VMEM((PPW,), jnp.int32),
        pltpu.VMEM((PPW,), jnp.int32),
        pltpu.VMEM((PPW, DIM), jnp.float32),
        pltpu.VMEM((PPW, DIM), jnp.float32),
        pltpu.VMEM((PPW,), jnp.float32),
        pltpu.SemaphoreType.DMA,
    ],
)
def _pairdot(e0, e1, e2, e3, users, items, out,
             uidx, iidx, ubuf, ibuf, gbuf, sem):
    c = lax.axis_index("c")
    s = lax.axis_index("s")
    w = s * NC + c
    pb = w * PPW
    pltpu.sync_copy(users.at[pl.ds(pb, PPW)], uidx)
    pltpu.sync_copy(items.at[pl.ds(pb, PPW)], iidx)

    def addoff(q, _):
        sl = pl.ds(q * 16, 16)
        iidx[sl] = iidx[sl] + N_USERS
        return 0

    lax.fori_loop(0, PPW // 16, addoff, 0)

    pltpu.async_copy(e0.at[uidx], ubuf, sem).wait()
    pltpu.async_copy(e1.at[uidx], ubuf, sem, add=True).wait()
    pltpu.async_copy(e2.at[uidx], ubuf, sem, add=True).wait()
    pltpu.async_copy(e3.at[uidx], ubuf, sem, add=True).wait()
    pltpu.async_copy(e0.at[iidx], ibuf, sem).wait()
    pltpu.async_copy(e1.at[iidx], ibuf, sem, add=True).wait()
    pltpu.async_copy(e2.at[iidx], ibuf, sem, add=True).wait()
    pltpu.async_copy(e3.at[iidx], ibuf, sem, add=True).wait()

    lanes = lax.broadcasted_iota(jnp.int32, (16,), 0)

    def gloop(g, _):
        rows16 = lanes + g * 16

        def dloop(d, a):
            dd = jnp.full((16,), 0, jnp.int32) + d
            uu = plsc.load_gather(ubuf, [rows16, dd])
            ii = plsc.load_gather(ibuf, [rows16, dd])
            return a + uu * ii

        a = lax.fori_loop(0, DIM, dloop, jnp.zeros((16,), jnp.float32))
        gbuf[pl.ds(g * 16, 16)] = a * 0.0625
        return 0

    lax.fori_loop(0, PPW // 16, gloop, 0)
    pltpu.sync_copy(gbuf, out.at[pl.ds(pb, PPW)])


def kernel(users, items, edge_index, edge_vals, user_emb, item_emb):
    all0 = jnp.concatenate([user_emb, item_emb], axis=0)
    dst = edge_index[0]
    src = edge_index[1]
    zeros = jnp.zeros((RPW, DIM), jnp.float32)
    e1 = _spmm(all0, src, dst, edge_vals, zeros)
    e2 = _spmm(e1, src, dst, edge_vals, zeros)
    e3 = _spmm(e2, src, dst, edge_vals, zeros)
    return _pairdot(all0, e1, e2, e3, users, items)
